# baseline (speedup 1.0000x reference)
import jax
import jax.numpy as jnp
from jax import lax
from jax.experimental import pallas as pl
from jax.experimental.pallas import tpu as pltpu
from jax.experimental.pallas import tpu_sc as plsc

N = 10000
E = 320000
D = 128
DH = D // 2
EPS = 1e-09

NC = 2
NS = 16
K = 128
NBUF = 4
SLAB = 80
PHASES = 2
CHUNKS_TILE = SLAB * PHASES
E_PER_TILE = CHUNKS_TILE * K
E_PAD = E_PER_TILE * NS
ROWS_PER_TILE = 632
NPAD = ROWS_PER_TILE * NS

BN = 400
GRID = N // BN


def _tc_a_body(x_ref, w0_ref, w1_ref, b0_ref, att0_ref, att1_ref, attb_ref,
               sc0_ref, off0_ref, rs_ref, vwn_ref, att_ref):
    x = x_ref[:]
    h0 = jnp.dot(x, w0_ref[:], preferred_element_type=jnp.float32)
    h0 = jnp.maximum(h0 + b0_ref[:], 0.0)
    m = jnp.mean(h0, axis=1, keepdims=True)
    v = jnp.mean((h0 - m) ** 2, axis=1, keepdims=True)
    rs_ref[:] = sc0_ref[:] * (h0 - m) * lax.rsqrt(v + EPS) + off0_ref[:]
    h1 = jnp.dot(x, w1_ref[:], preferred_element_type=jnp.float32)
    vwn_ref[0] = h1[:, :DH]
    vwn_ref[1] = h1[:, DH:]
    a_n = lax.dot_general(att1_ref[:], h1, (((1,), (1,)), ((), ())),
                          precision=lax.Precision.HIGHEST,
                          preferred_element_type=jnp.float32) + attb_ref[0, 1]
    a_s = lax.dot_general(att0_ref[:], h1, (((1,), (1,)), ((), ())),
                          precision=lax.Precision.HIGHEST,
                          preferred_element_type=jnp.float32) + attb_ref[0, 0]
    att_ref[0] = jnp.concatenate([a_n, a_s], axis=0)


def _tc_a(vecs, w0, w1, b0, att0, att1, attb, sc0, off0):
    full = lambda i: (0, 0)
    return pl.pallas_call(
        _tc_a_body,
        grid=(GRID,),
        in_specs=[
            pl.BlockSpec((BN, D), lambda i: (i, 0)),
            pl.BlockSpec((D, D), full),
            pl.BlockSpec((D, D), full),
            pl.BlockSpec((1, D), full),
            pl.BlockSpec((1, D), full),
            pl.BlockSpec((1, D), full),
            pl.BlockSpec(memory_space=pltpu.SMEM),
            pl.BlockSpec((1, D), full),
            pl.BlockSpec((1, D), full),
        ],
        out_specs=[
            pl.BlockSpec((BN, D), lambda i: (i, 0)),
            pl.BlockSpec((NC, BN, DH), lambda i: (0, i, 0)),
            pl.BlockSpec((1, 2, BN), lambda i: (i, 0, 0)),
        ],
        out_shape=[
            jax.ShapeDtypeStruct((N, D), jnp.float32),
            jax.ShapeDtypeStruct((NC, N, DH), jnp.float32),
            jax.ShapeDtypeStruct((GRID, 2, BN), jnp.float32),
        ],
    )(vecs, w0, w1, b0, att0, att1, attb, sc0, off0)


def _sc_body(row2_hbm, col2_hbm, vals2_hbm, an_hbm, as_hbm, vwnf_hbm,
             zeros_hbm, out_hbm,
             an_v, as_v, row_a, col_a, w_a, rows0, rows1, rows2, rows3,
             agg_sh, g0, g1, g2, g3, s0, s1, s2, s3):
    c = lax.axis_index("c")
    s = lax.axis_index("s")
    c_n = c * N
    bufs = [rows0, rows1, rows2, rows3]
    gsems = [g0, g1, g2, g3]
    ssems = [s0, s1, s2, s3]

    pltpu.sync_copy(zeros_hbm, agg_sh.at[pl.ds(s * ROWS_PER_TILE, ROWS_PER_TILE)])
    pltpu.sync_copy(an_hbm, an_v)
    pltpu.sync_copy(as_hbm, as_v)

    plsc.subcore_barrier()

    lanes = [jnp.full((16,), l, jnp.int32) for l in range(16)]

    def mult(buf, k):
        def mbody(i, carry):
            w16 = w_a[k, pl.ds(i * 16, 16)]
            for l in range(16):
                wb = w16.at[lanes[l]].get(mode="promise_in_bounds")
                e = i * 16 + l
                for j in range(DH // 16):
                    rsl = pl.ds(j * 16, 16)
                    buf[e, rsl] = buf[e, rsl] * wb
            return carry

        lax.fori_loop(0, K // 16, mbody, 0)

    for p in range(PHASES):
        base = pl.multiple_of(s * CHUNKS_TILE + p * SLAB, SLAB)
        pltpu.sync_copy(row2_hbm.at[pl.ds(base, SLAB)], row_a)
        pltpu.sync_copy(col2_hbm.at[pl.ds(base, SLAB)], col_a)
        pltpu.sync_copy(vals2_hbm.at[pl.ds(base, SLAB)], w_a)

        def wbody(k, carry):
            for i in range(K // 16):
                sl = pl.ds(i * 16, 16)
                col16 = col_a[k, sl]
                a1 = plsc.load_gather(an_v, [col16])
                a2 = plsc.load_gather(as_v, [row_a[k, sl]])
                w_a[k, sl] = w_a[k, sl] * jnp.maximum(a1 + a2, 0.0)
                col_a[k, sl] = col16 + c_n
            return carry

        lax.fori_loop(0, SLAB, wbody, 0)

        pltpu.async_copy(vwnf_hbm.at[col_a.at[0]], bufs[0], gsems[0])
        pltpu.async_copy(vwnf_hbm.at[col_a.at[1]], bufs[1], gsems[1])

        def pipe(t, carry):
            for b in range(NBUF):
                k = NBUF * t + b
                pltpu.make_async_copy(vwnf_hbm.at[col_a.at[k]], bufs[b],
                                      gsems[b]).wait()
                mult(bufs[b], k)
                pltpu.async_copy(bufs[b], agg_sh.at[row_a.at[k]], ssems[b],
                                 add=True)
                br = (b + 2) % NBUF

                @pl.when(k + 2 < SLAB)
                def _prefetch():
                    @pl.when(k >= 2)
                    def _drain_prev():
                        pltpu.make_async_copy(
                            bufs[br], agg_sh.at[row_a.at[k - 2]],
                            ssems[br]).wait()

                    pltpu.async_copy(vwnf_hbm.at[col_a.at[k + 2]], bufs[br],
                                     gsems[br])

            return carry

        lax.fori_loop(0, SLAB // NBUF, pipe, 0)

        for b in range(NBUF):
            kt = SLAB - NBUF + b
            pltpu.make_async_copy(bufs[b], agg_sh.at[row_a.at[kt]],
                                  ssems[b]).wait()

    plsc.subcore_barrier()
    pltpu.sync_copy(agg_sh.at[pl.ds(s * ROWS_PER_TILE, ROWS_PER_TILE)],
                    out_hbm.at[c, pl.ds(s * ROWS_PER_TILE, ROWS_PER_TILE)])


def _sc_aggregate(row2, col2, vals2, a_n, a_s, vwnf, zeros_rows):
    mesh = plsc.VectorSubcoreMesh(core_axis_name="c", subcore_axis_name="s")
    f = pl.kernel(
        _sc_body,
        out_type=jax.ShapeDtypeStruct((NC, NPAD, DH), jnp.float32),
        mesh=mesh,
        scratch_types=[
            pltpu.VMEM((N,), jnp.float32),
            pltpu.VMEM((N,), jnp.float32),
            pltpu.VMEM((SLAB, K), jnp.int32),
            pltpu.VMEM((SLAB, K), jnp.int32),
            pltpu.VMEM((SLAB, K), jnp.float32),
            pltpu.VMEM((K, DH), jnp.float32),
            pltpu.VMEM((K, DH), jnp.float32),
            pltpu.VMEM((K, DH), jnp.float32),
            pltpu.VMEM((K, DH), jnp.float32),
            pltpu.VMEM_SHARED((NPAD, DH), jnp.float32),
            pltpu.SemaphoreType.DMA,
            pltpu.SemaphoreType.DMA,
            pltpu.SemaphoreType.DMA,
            pltpu.SemaphoreType.DMA,
            pltpu.SemaphoreType.DMA,
            pltpu.SemaphoreType.DMA,
            pltpu.SemaphoreType.DMA,
            pltpu.SemaphoreType.DMA,
        ],
        compiler_params=pltpu.CompilerParams(needs_layout_passes=False,
                                             use_tc_tiling_on_sc=False),
    )
    return f(row2, col2, vals2, a_n, a_s, vwnf, zeros_rows)


def _tc_c_body(p_ref, rs_ref, b1_ref, sc1_ref, off1_ref, out_ref):
    agg = jnp.concatenate([p_ref[0], p_ref[1]], axis=1)
    rn = jnp.maximum(agg, 0.0) + b1_ref[:]
    m = jnp.mean(rn, axis=1, keepdims=True)
    v = jnp.mean((rn - m) ** 2, axis=1, keepdims=True)
    out_ref[:] = (sc1_ref[:] * (rn - m) * lax.rsqrt(v + EPS) + off1_ref[:]
                  + rs_ref[:])


def _tc_c(partials, ret_self, b1, sc1, off1):
    full = lambda i: (0, 0)
    return pl.pallas_call(
        _tc_c_body,
        grid=(GRID,),
        in_specs=[
            pl.BlockSpec((NC, BN, DH), lambda i: (0, i, 0)),
            pl.BlockSpec((BN, D), lambda i: (i, 0)),
            pl.BlockSpec((1, D), full),
            pl.BlockSpec((1, D), full),
            pl.BlockSpec((1, D), full),
        ],
        out_specs=pl.BlockSpec((BN, D), lambda i: (i, 0)),
        out_shape=jax.ShapeDtypeStruct((N, D), jnp.float32),
    )(partials, ret_self, b1, sc1, off1)


def kernel(vecs, edge_index, edge_vals, W0, b0, W1, b1, att0, att1,
           att_b0, att_b1, off0, sc0, off1, sc1):
    b0r = b0.reshape(1, D)
    b1r = b1.reshape(1, D)
    attb = jnp.concatenate([att_b0, att_b1]).reshape(1, 2)

    ret_self, vwn, att = _tc_a(vecs, W0, W1, b0r, att0, att1, attb, sc0, off0)
    a_n = att[:, 0, :].reshape(N)
    a_s = att[:, 1, :].reshape(N)
    vwnf = vwn.reshape(NC * N, DH)

    pad = E_PAD - E
    row = jnp.concatenate([edge_index[0], jnp.zeros((pad,), jnp.int32)])
    col = jnp.concatenate([edge_index[1], jnp.zeros((pad,), jnp.int32)])
    vals = jnp.concatenate([edge_vals, jnp.zeros((pad,), jnp.float32)])
    row2 = row.reshape(E_PAD // K, K)
    col2 = col.reshape(E_PAD // K, K)
    vals2 = vals.reshape(E_PAD // K, K)
    zeros_rows = jnp.zeros((ROWS_PER_TILE, DH), jnp.float32)

    partials = _sc_aggregate(row2, col2, vals2, a_n, a_s, vwnf, zeros_rows)

    return _tc_c(partials, ret_self, b1r, sc1, off1)


# speedup vs baseline: 1.3085x; 1.3085x over previous
import jax
import jax.numpy as jnp
from jax import lax
from jax.experimental import pallas as pl
from jax.experimental.pallas import tpu as pltpu
from jax.experimental.pallas import tpu_sc as plsc

N = 10000
E = 320000
D = 128
DH = D // 2
EPS = 1e-09

NC = 2
NS = 16
K = 128
NBUF = 4
SLAB = 80
PHASES = 2
CHUNKS_TILE = SLAB * PHASES
E_PER_TILE = CHUNKS_TILE * K
E_PAD = E_PER_TILE * NS
ROWS_PER_TILE = 632
NPAD = ROWS_PER_TILE * NS

BN = 400
GRID = N // BN


def _tc_a_body(x_ref, w0_ref, w1_ref, b0_ref, att0_ref, att1_ref, attb_ref,
               sc0_ref, off0_ref, rs_ref, vwn_ref, att_ref):
    x = x_ref[:]
    h0 = jnp.dot(x, w0_ref[:], preferred_element_type=jnp.float32)
    h0 = jnp.maximum(h0 + b0_ref[:], 0.0)
    m = jnp.mean(h0, axis=1, keepdims=True)
    v = jnp.mean((h0 - m) ** 2, axis=1, keepdims=True)
    rs_ref[:] = sc0_ref[:] * (h0 - m) * lax.rsqrt(v + EPS) + off0_ref[:]
    h1 = jnp.dot(x, w1_ref[:], preferred_element_type=jnp.float32)
    vwn_ref[0] = h1[:, :DH]
    vwn_ref[1] = h1[:, DH:]
    a_n = lax.dot_general(att1_ref[:], h1, (((1,), (1,)), ((), ())),
                          precision=lax.Precision.HIGHEST,
                          preferred_element_type=jnp.float32) + attb_ref[0, 1]
    a_s = lax.dot_general(att0_ref[:], h1, (((1,), (1,)), ((), ())),
                          precision=lax.Precision.HIGHEST,
                          preferred_element_type=jnp.float32) + attb_ref[0, 0]
    att_ref[0] = jnp.concatenate([a_n, a_s], axis=0)


def _tc_a(vecs, w0, w1, b0, att0, att1, attb, sc0, off0):
    full = lambda i: (0, 0)
    return pl.pallas_call(
        _tc_a_body,
        grid=(GRID,),
        in_specs=[
            pl.BlockSpec((BN, D), lambda i: (i, 0)),
            pl.BlockSpec((D, D), full),
            pl.BlockSpec((D, D), full),
            pl.BlockSpec((1, D), full),
            pl.BlockSpec((1, D), full),
            pl.BlockSpec((1, D), full),
            pl.BlockSpec(memory_space=pltpu.SMEM),
            pl.BlockSpec((1, D), full),
            pl.BlockSpec((1, D), full),
        ],
        out_specs=[
            pl.BlockSpec((BN, D), lambda i: (i, 0)),
            pl.BlockSpec((NC, BN, DH), lambda i: (0, i, 0)),
            pl.BlockSpec((1, 2, BN), lambda i: (i, 0, 0)),
        ],
        out_shape=[
            jax.ShapeDtypeStruct((N, D), jnp.float32),
            jax.ShapeDtypeStruct((NC, N, DH), jnp.float32),
            jax.ShapeDtypeStruct((GRID, 2, BN), jnp.float32),
        ],
    )(vecs, w0, w1, b0, att0, att1, attb, sc0, off0)


def _sc_body(row2_hbm, col2_hbm, vals2_hbm, an_hbm, as_hbm, vwnf_hbm,
             zeros_hbm, out_hbm,
             an_v, as_v, row_a, col_a, w_a, rows0, rows1, rows2, rows3,
             agg_sh, g0, g1, g2, g3, s0, s1, s2, s3):
    c = lax.axis_index("c")
    s = lax.axis_index("s")
    c_n = c * N
    bufs = [rows0, rows1, rows2, rows3]
    gsems = [g0, g1, g2, g3]
    ssems = [s0, s1, s2, s3]

    pltpu.sync_copy(zeros_hbm, agg_sh.at[pl.ds(s * ROWS_PER_TILE, ROWS_PER_TILE)])
    pltpu.sync_copy(an_hbm, an_v)
    pltpu.sync_copy(as_hbm, as_v)

    plsc.subcore_barrier()

    def mult(buf, k):
        @plsc.parallel_loop(0, K // 16)
        def mbody(i):
            w16 = w_a[k, pl.ds(i * 16, 16)]
            for l in range(16):
                we = w16[l]
                e = i * 16 + l
                for j in range(DH // 16):
                    rsl = pl.ds(j * 16, 16)
                    buf[e, rsl] = buf[e, rsl] * we

    for p in range(PHASES):
        base = pl.multiple_of(s * CHUNKS_TILE + p * SLAB, SLAB)
        pltpu.sync_copy(row2_hbm.at[pl.ds(base, SLAB)], row_a)
        pltpu.sync_copy(col2_hbm.at[pl.ds(base, SLAB)], col_a)
        pltpu.sync_copy(vals2_hbm.at[pl.ds(base, SLAB)], w_a)

        def wbody(k, carry):
            for i in range(K // 16):
                sl = pl.ds(i * 16, 16)
                col16 = col_a[k, sl]
                a1 = plsc.load_gather(an_v, [col16])
                a2 = plsc.load_gather(as_v, [row_a[k, sl]])
                w_a[k, sl] = w_a[k, sl] * jnp.maximum(a1 + a2, 0.0)
                col_a[k, sl] = col16 + c_n
            return carry

        lax.fori_loop(0, SLAB, wbody, 0)

        pltpu.async_copy(vwnf_hbm.at[col_a.at[0]], bufs[0], gsems[0])
        pltpu.async_copy(vwnf_hbm.at[col_a.at[1]], bufs[1], gsems[1])

        def pipe(t, carry):
            for b in range(NBUF):
                k = NBUF * t + b
                pltpu.make_async_copy(vwnf_hbm.at[col_a.at[k]], bufs[b],
                                      gsems[b]).wait()
                mult(bufs[b], k)
                pltpu.async_copy(bufs[b], agg_sh.at[row_a.at[k]], ssems[b],
                                 add=True)
                br = (b + 2) % NBUF

                @pl.when(k + 2 < SLAB)
                def _prefetch():
                    @pl.when(k >= 2)
                    def _drain_prev():
                        pltpu.make_async_copy(
                            bufs[br], agg_sh.at[row_a.at[k - 2]],
                            ssems[br]).wait()

                    pltpu.async_copy(vwnf_hbm.at[col_a.at[k + 2]], bufs[br],
                                     gsems[br])

            return carry

        lax.fori_loop(0, SLAB // NBUF, pipe, 0)

        for b in range(NBUF):
            kt = SLAB - NBUF + b
            pltpu.make_async_copy(bufs[b], agg_sh.at[row_a.at[kt]],
                                  ssems[b]).wait()

    plsc.subcore_barrier()
    pltpu.sync_copy(agg_sh.at[pl.ds(s * ROWS_PER_TILE, ROWS_PER_TILE)],
                    out_hbm.at[c, pl.ds(s * ROWS_PER_TILE, ROWS_PER_TILE)])


def _sc_aggregate(row2, col2, vals2, a_n, a_s, vwnf, zeros_rows):
    mesh = plsc.VectorSubcoreMesh(core_axis_name="c", subcore_axis_name="s")
    f = pl.kernel(
        _sc_body,
        out_type=jax.ShapeDtypeStruct((NC, NPAD, DH), jnp.float32),
        mesh=mesh,
        scratch_types=[
            pltpu.VMEM((N,), jnp.float32),
            pltpu.VMEM((N,), jnp.float32),
            pltpu.VMEM((SLAB, K), jnp.int32),
            pltpu.VMEM((SLAB, K), jnp.int32),
            pltpu.VMEM((SLAB, K), jnp.float32),
            pltpu.VMEM((K, DH), jnp.float32),
            pltpu.VMEM((K, DH), jnp.float32),
            pltpu.VMEM((K, DH), jnp.float32),
            pltpu.VMEM((K, DH), jnp.float32),
            pltpu.VMEM_SHARED((NPAD, DH), jnp.float32),
            pltpu.SemaphoreType.DMA,
            pltpu.SemaphoreType.DMA,
            pltpu.SemaphoreType.DMA,
            pltpu.SemaphoreType.DMA,
            pltpu.SemaphoreType.DMA,
            pltpu.SemaphoreType.DMA,
            pltpu.SemaphoreType.DMA,
            pltpu.SemaphoreType.DMA,
        ],
        compiler_params=pltpu.CompilerParams(needs_layout_passes=False,
                                             use_tc_tiling_on_sc=False),
    )
    return f(row2, col2, vals2, a_n, a_s, vwnf, zeros_rows)


def _tc_c_body(p_ref, rs_ref, b1_ref, sc1_ref, off1_ref, out_ref):
    agg = jnp.concatenate([p_ref[0], p_ref[1]], axis=1)
    rn = jnp.maximum(agg, 0.0) + b1_ref[:]
    m = jnp.mean(rn, axis=1, keepdims=True)
    v = jnp.mean((rn - m) ** 2, axis=1, keepdims=True)
    out_ref[:] = (sc1_ref[:] * (rn - m) * lax.rsqrt(v + EPS) + off1_ref[:]
                  + rs_ref[:])


def _tc_c(partials, ret_self, b1, sc1, off1):
    full = lambda i: (0, 0)
    return pl.pallas_call(
        _tc_c_body,
        grid=(GRID,),
        in_specs=[
            pl.BlockSpec((NC, BN, DH), lambda i: (0, i, 0)),
            pl.BlockSpec((BN, D), lambda i: (i, 0)),
            pl.BlockSpec((1, D), full),
            pl.BlockSpec((1, D), full),
            pl.BlockSpec((1, D), full),
        ],
        out_specs=pl.BlockSpec((BN, D), lambda i: (i, 0)),
        out_shape=jax.ShapeDtypeStruct((N, D), jnp.float32),
    )(partials, ret_self, b1, sc1, off1)


def kernel(vecs, edge_index, edge_vals, W0, b0, W1, b1, att0, att1,
           att_b0, att_b1, off0, sc0, off1, sc1):
    b0r = b0.reshape(1, D)
    b1r = b1.reshape(1, D)
    attb = jnp.concatenate([att_b0, att_b1]).reshape(1, 2)

    ret_self, vwn, att = _tc_a(vecs, W0, W1, b0r, att0, att1, attb, sc0, off0)
    a_n = att[:, 0, :].reshape(N)
    a_s = att[:, 1, :].reshape(N)
    vwnf = vwn.reshape(NC * N, DH)

    pad = E_PAD - E
    row = jnp.concatenate([edge_index[0], jnp.zeros((pad,), jnp.int32)])
    col = jnp.concatenate([edge_index[1], jnp.zeros((pad,), jnp.int32)])
    vals = jnp.concatenate([edge_vals, jnp.zeros((pad,), jnp.float32)])
    row2 = row.reshape(E_PAD // K, K)
    col2 = col.reshape(E_PAD // K, K)
    vals2 = vals.reshape(E_PAD // K, K)
    zeros_rows = jnp.zeros((ROWS_PER_TILE, DH), jnp.float32)

    partials = _sc_aggregate(row2, col2, vals2, a_n, a_s, vwnf, zeros_rows)

    return _tc_c(partials, ret_self, b1r, sc1, off1)


# speedup vs baseline: 1.3404x; 1.0244x over previous
import jax
import jax.numpy as jnp
from jax import lax
from jax.experimental import pallas as pl
from jax.experimental.pallas import tpu as pltpu
from jax.experimental.pallas import tpu_sc as plsc

N = 10000
E = 320000
D = 128
DH = D // 2
EPS = 1e-09

NC = 2
NS = 16
K = 128
NBUF = 4
SLAB = 80
PHASES = 2
CHUNKS_TILE = SLAB * PHASES
E_PER_TILE = CHUNKS_TILE * K
E_PAD = E_PER_TILE * NS
ROWS_PER_TILE = 632
NPAD = ROWS_PER_TILE * NS

BN = 400
GRID = N // BN


def _tc_a_body(x_ref, w0_ref, w1_ref, b0_ref, att0_ref, att1_ref, attb_ref,
               sc0_ref, off0_ref, rs_ref, vwn_ref, att_ref):
    x = x_ref[:]
    h0 = jnp.dot(x, w0_ref[:], preferred_element_type=jnp.float32)
    h0 = jnp.maximum(h0 + b0_ref[:], 0.0)
    m = jnp.mean(h0, axis=1, keepdims=True)
    v = jnp.mean((h0 - m) ** 2, axis=1, keepdims=True)
    rs_ref[:] = sc0_ref[:] * (h0 - m) * lax.rsqrt(v + EPS) + off0_ref[:]
    h1 = jnp.dot(x, w1_ref[:], preferred_element_type=jnp.float32)
    vwn_ref[0] = h1[:, :DH]
    vwn_ref[1] = h1[:, DH:]
    a_n = lax.dot_general(att1_ref[:], h1, (((1,), (1,)), ((), ())),
                          precision=lax.Precision.HIGHEST,
                          preferred_element_type=jnp.float32) + attb_ref[0, 1]
    a_s = lax.dot_general(att0_ref[:], h1, (((1,), (1,)), ((), ())),
                          precision=lax.Precision.HIGHEST,
                          preferred_element_type=jnp.float32) + attb_ref[0, 0]
    att_ref[0] = jnp.concatenate([a_n, a_s], axis=0)


def _tc_a(vecs, w0, w1, b0, att0, att1, attb, sc0, off0):
    full = lambda i: (0, 0)
    return pl.pallas_call(
        _tc_a_body,
        grid=(GRID,),
        in_specs=[
            pl.BlockSpec((BN, D), lambda i: (i, 0)),
            pl.BlockSpec((D, D), full),
            pl.BlockSpec((D, D), full),
            pl.BlockSpec((1, D), full),
            pl.BlockSpec((1, D), full),
            pl.BlockSpec((1, D), full),
            pl.BlockSpec(memory_space=pltpu.SMEM),
            pl.BlockSpec((1, D), full),
            pl.BlockSpec((1, D), full),
        ],
        out_specs=[
            pl.BlockSpec((BN, D), lambda i: (i, 0)),
            pl.BlockSpec((NC, BN, DH), lambda i: (0, i, 0)),
            pl.BlockSpec((1, 2, BN), lambda i: (i, 0, 0)),
        ],
        out_shape=[
            jax.ShapeDtypeStruct((N, D), jnp.float32),
            jax.ShapeDtypeStruct((NC, N, DH), jnp.float32),
            jax.ShapeDtypeStruct((GRID, 2, BN), jnp.float32),
        ],
    )(vecs, w0, w1, b0, att0, att1, attb, sc0, off0)


def _sc_body(row2_hbm, col2_hbm, vals2_hbm, an_hbm, as_hbm, vwnf_hbm,
             zeros_hbm, out_hbm,
             an_v, as_v, row_a, col_a, w_a, rows0, rows1, rows2, rows3,
             agg_sh, g0, g1, g2, g3, s0, s1, s2, s3):
    c = lax.axis_index("c")
    s = lax.axis_index("s")
    c_n = c * N
    bufs = [rows0, rows1, rows2, rows3]
    gsems = [g0, g1, g2, g3]
    ssems = [s0, s1, s2, s3]

    pltpu.sync_copy(zeros_hbm, agg_sh.at[pl.ds(s * ROWS_PER_TILE, ROWS_PER_TILE)])
    pltpu.sync_copy(an_hbm, an_v)
    pltpu.sync_copy(as_hbm, as_v)

    plsc.subcore_barrier()

    def mult(buf, k):
        @plsc.parallel_loop(0, K // 16)
        def mbody(i):
            w16 = w_a[k, pl.ds(i * 16, 16)]
            for l in range(16):
                we = w16[l]
                e = i * 16 + l
                for j in range(DH // 16):
                    rsl = pl.ds(j * 16, 16)
                    buf[e, rsl] = buf[e, rsl] * we

    for p in range(PHASES):
        base = pl.multiple_of(s * CHUNKS_TILE + p * SLAB, SLAB)
        pltpu.sync_copy(row2_hbm.at[pl.ds(base, SLAB)], row_a)
        pltpu.sync_copy(col2_hbm.at[pl.ds(base, SLAB)], col_a)
        pltpu.sync_copy(vals2_hbm.at[pl.ds(base, SLAB)], w_a)

        @plsc.parallel_loop(0, SLAB)
        def wbody(k):
            for i in range(K // 16):
                sl = pl.ds(i * 16, 16)
                col16 = col_a[k, sl]
                a1 = plsc.load_gather(an_v, [col16])
                a2 = plsc.load_gather(as_v, [row_a[k, sl]])
                w_a[k, sl] = w_a[k, sl] * jnp.maximum(a1 + a2, 0.0)
                col_a[k, sl] = col16 + c_n

        pltpu.async_copy(vwnf_hbm.at[col_a.at[0]], bufs[0], gsems[0])
        pltpu.async_copy(vwnf_hbm.at[col_a.at[1]], bufs[1], gsems[1])

        def pipe(t, carry):
            for b in range(NBUF):
                k = NBUF * t + b
                pltpu.make_async_copy(vwnf_hbm.at[col_a.at[k]], bufs[b],
                                      gsems[b]).wait()
                mult(bufs[b], k)
                pltpu.async_copy(bufs[b], agg_sh.at[row_a.at[k]], ssems[b],
                                 add=True)
                br = (b + 2) % NBUF

                @pl.when(k + 2 < SLAB)
                def _prefetch():
                    @pl.when(k >= 2)
                    def _drain_prev():
                        pltpu.make_async_copy(
                            bufs[br], agg_sh.at[row_a.at[k - 2]],
                            ssems[br]).wait()

                    pltpu.async_copy(vwnf_hbm.at[col_a.at[k + 2]], bufs[br],
                                     gsems[br])

            return carry

        lax.fori_loop(0, SLAB // NBUF, pipe, 0)

        for b in range(NBUF):
            kt = SLAB - NBUF + b
            pltpu.make_async_copy(bufs[b], agg_sh.at[row_a.at[kt]],
                                  ssems[b]).wait()

    plsc.subcore_barrier()
    pltpu.sync_copy(agg_sh.at[pl.ds(s * ROWS_PER_TILE, ROWS_PER_TILE)],
                    out_hbm.at[c, pl.ds(s * ROWS_PER_TILE, ROWS_PER_TILE)])


def _sc_aggregate(row2, col2, vals2, a_n, a_s, vwnf, zeros_rows):
    mesh = plsc.VectorSubcoreMesh(core_axis_name="c", subcore_axis_name="s")
    f = pl.kernel(
        _sc_body,
        out_type=jax.ShapeDtypeStruct((NC, NPAD, DH), jnp.float32),
        mesh=mesh,
        scratch_types=[
            pltpu.VMEM((N,), jnp.float32),
            pltpu.VMEM((N,), jnp.float32),
            pltpu.VMEM((SLAB, K), jnp.int32),
            pltpu.VMEM((SLAB, K), jnp.int32),
            pltpu.VMEM((SLAB, K), jnp.float32),
            pltpu.VMEM((K, DH), jnp.float32),
            pltpu.VMEM((K, DH), jnp.float32),
            pltpu.VMEM((K, DH), jnp.float32),
            pltpu.VMEM((K, DH), jnp.float32),
            pltpu.VMEM_SHARED((NPAD, DH), jnp.float32),
            pltpu.SemaphoreType.DMA,
            pltpu.SemaphoreType.DMA,
            pltpu.SemaphoreType.DMA,
            pltpu.SemaphoreType.DMA,
            pltpu.SemaphoreType.DMA,
            pltpu.SemaphoreType.DMA,
            pltpu.SemaphoreType.DMA,
            pltpu.SemaphoreType.DMA,
        ],
        compiler_params=pltpu.CompilerParams(needs_layout_passes=False,
                                             use_tc_tiling_on_sc=False),
    )
    return f(row2, col2, vals2, a_n, a_s, vwnf, zeros_rows)


def _tc_c_body(p_ref, rs_ref, b1_ref, sc1_ref, off1_ref, out_ref):
    agg = jnp.concatenate([p_ref[0], p_ref[1]], axis=1)
    rn = jnp.maximum(agg, 0.0) + b1_ref[:]
    m = jnp.mean(rn, axis=1, keepdims=True)
    v = jnp.mean((rn - m) ** 2, axis=1, keepdims=True)
    out_ref[:] = (sc1_ref[:] * (rn - m) * lax.rsqrt(v + EPS) + off1_ref[:]
                  + rs_ref[:])


def _tc_c(partials, ret_self, b1, sc1, off1):
    full = lambda i: (0, 0)
    return pl.pallas_call(
        _tc_c_body,
        grid=(GRID,),
        in_specs=[
            pl.BlockSpec((NC, BN, DH), lambda i: (0, i, 0)),
            pl.BlockSpec((BN, D), lambda i: (i, 0)),
            pl.BlockSpec((1, D), full),
            pl.BlockSpec((1, D), full),
            pl.BlockSpec((1, D), full),
        ],
        out_specs=pl.BlockSpec((BN, D), lambda i: (i, 0)),
        out_shape=jax.ShapeDtypeStruct((N, D), jnp.float32),
    )(partials, ret_self, b1, sc1, off1)


def kernel(vecs, edge_index, edge_vals, W0, b0, W1, b1, att0, att1,
           att_b0, att_b1, off0, sc0, off1, sc1):
    b0r = b0.reshape(1, D)
    b1r = b1.reshape(1, D)
    attb = jnp.concatenate([att_b0, att_b1]).reshape(1, 2)

    ret_self, vwn, att = _tc_a(vecs, W0, W1, b0r, att0, att1, attb, sc0, off0)
    a_n = att[:, 0, :].reshape(N)
    a_s = att[:, 1, :].reshape(N)
    vwnf = vwn.reshape(NC * N, DH)

    pad = E_PAD - E
    row = jnp.concatenate([edge_index[0], jnp.zeros((pad,), jnp.int32)])
    col = jnp.concatenate([edge_index[1], jnp.zeros((pad,), jnp.int32)])
    vals = jnp.concatenate([edge_vals, jnp.zeros((pad,), jnp.float32)])
    row2 = row.reshape(E_PAD // K, K)
    col2 = col.reshape(E_PAD // K, K)
    vals2 = vals.reshape(E_PAD // K, K)
    zeros_rows = jnp.zeros((ROWS_PER_TILE, DH), jnp.float32)

    partials = _sc_aggregate(row2, col2, vals2, a_n, a_s, vwnf, zeros_rows)

    return _tc_c(partials, ret_self, b1r, sc1, off1)
